# Initial kernel scaffold; baseline (speedup 1.0000x reference)
import jax
import jax.numpy as jnp
from jax.experimental import pallas as pl


def kernel(x, edge_index, edge_weight, W1_rel, W1_root, b1, W2_rel, W2_root, b2):
    raise NotImplementedError("write your pallas kernel here")


# speedup vs baseline: 8.4944x; 8.4944x over previous
import functools

import jax
import jax.numpy as jnp
from jax import lax
from jax.experimental import pallas as pl
from jax.experimental.pallas import tpu as pltpu
from jax.experimental.pallas import tpu_sc as plsc

LANE = 16


def kernel(x, edge_index, edge_weight, W1_rel, W1_root, b1, W2_rel, W2_root, b2):
    N, D = x.shape
    H = W1_rel.shape[1]
    C = W2_rel.shape[1]
    E = edge_index.shape[1]
    HP = 16
    NTILES = 32

    per_tile = -(-E // NTILES)
    per_tile = -(-per_tile // 1024) * 1024
    EP = per_tile * NTILES
    NR = per_tile // 128
    NROUT = NR // 8

    NACC = -(-(N + 16) // 128) * 128
    RPT = NACC // 16

    pad = EP - E
    src = jnp.concatenate([edge_index[0], jnp.zeros((pad,), jnp.int32)])
    dst = jnp.concatenate([edge_index[1], jnp.full((pad,), N, jnp.int32)])
    wgt = jnp.concatenate([edge_weight, jnp.zeros((pad,), jnp.float32)])
    src2d = src.reshape(EP // 128, 128)
    dst2d = dst.reshape(EP // 128, 128)
    w2d = wgt.reshape(EP // 128, 128)
    cpat = jnp.zeros((128, HP), jnp.float32).at[:, 0].set(1.0)

    bn = 1000
    Wcat = jnp.concatenate([W1_rel, W1_root], axis=1)
    bcat = jnp.concatenate([jnp.zeros((H,), jnp.float32), b1]).reshape(1, 2 * H)

    def a_body(x_ref, w_ref, b_ref, p1_ref, r1_ref):
        xw = jnp.dot(x_ref[...], w_ref[...], preferred_element_type=jnp.float32)
        xw = xw + b_ref[...]
        p1_ref[...] = xw[:, :H]
        r1_ref[...] = xw[:, H:]

    p1, r1 = pl.pallas_call(
        a_body,
        grid=(N // bn,),
        in_specs=[
            pl.BlockSpec((bn, D), lambda i: (i, 0)),
            pl.BlockSpec((D, 2 * H), lambda i: (0, 0)),
            pl.BlockSpec((1, 2 * H), lambda i: (0, 0)),
        ],
        out_specs=[
            pl.BlockSpec((bn, H), lambda i: (i, 0)),
            pl.BlockSpec((bn, H), lambda i: (i, 0)),
        ],
        out_shape=[
            jax.ShapeDtypeStruct((N, H), jnp.float32),
            jax.ShapeDtypeStruct((N, H), jnp.float32),
        ],
    )(x, Wcat, bcat)

    mesh = plsc.VectorSubcoreMesh(core_axis_name="c", subcore_axis_name="s")

    @functools.partial(
        pl.kernel,
        mesh=mesh,
        compiler_params=pltpu.CompilerParams(use_tc_tiling_on_sc=False),
        out_type=[
            jax.ShapeDtypeStruct((2, NACC, H), jnp.float32),
            jax.ShapeDtypeStruct((2, NACC, HP), jnp.float32),
        ],
        scratch_types=[
            pltpu.VMEM((8, 128), jnp.int32),
            pltpu.VMEM((8, 128), jnp.int32),
            pltpu.VMEM((1024,), jnp.float32),
            pltpu.VMEM((128, H), jnp.float32),
            pltpu.VMEM((128, HP), jnp.float32),
            pltpu.VMEM((RPT, H), jnp.float32),
            pltpu.VMEM((RPT, HP), jnp.float32),
            pltpu.VMEM_SHARED((NACC, H), jnp.float32),
            pltpu.VMEM_SHARED((NACC, HP), jnp.float32),
            pltpu.SemaphoreType.DMA,
        ],
    )
    def sc_aggregate1(p1_hbm, src_hbm, dst_hbm, w_hbm, cpat_hbm,
                      acc_out, cnt_out,
                      srcbuf, dstbuf, wbuf, rows, cpay, zbuf, zbuf16,
                      acc, cacc, sem):
        cid = lax.axis_index("c")
        sid = lax.axis_index("s")
        wid = cid * 16 + sid
        z16 = jnp.zeros((LANE,), jnp.float32)

        def zb(i, carry):
            zbuf[i, pl.ds(0, LANE)] = z16
            zbuf[i, pl.ds(LANE, LANE)] = z16
            zbuf16[i, pl.ds(0, LANE)] = z16
            return carry

        lax.fori_loop(0, RPT, zb, 0)
        r0 = sid * RPT
        pltpu.sync_copy(zbuf, acc.at[pl.ds(r0, RPT)])
        pltpu.sync_copy(zbuf16, cacc.at[pl.ds(r0, RPT)])
        pltpu.sync_copy(cpat_hbm, cpay)
        plsc.subcore_barrier()

        rb = wid * NR

        def chunk(k, carry):
            pltpu.sync_copy(src_hbm.at[pl.ds(rb + k * 8, 8)], srcbuf)
            pltpu.sync_copy(dst_hbm.at[pl.ds(rb + k * 8, 8)], dstbuf)
            pltpu.sync_copy(w_hbm.at[pl.ds((rb + k * 8) * 128, 1024)], wbuf)
            for j in range(8):
                pltpu.async_copy(p1_hbm.at[srcbuf.at[j]], rows, sem).wait()

                def gbody(g, c2):
                    wv = wbuf[pl.ds(j * 128 + g * LANE, LANE)]
                    for t in range(LANE):
                        e = g * LANE + t
                        we = lax.gather(
                            wv, jnp.full((LANE, 1), t, jnp.int32),
                            lax.GatherDimensionNumbers(
                                offset_dims=(), collapsed_slice_dims=(0,),
                                start_index_map=(0,)),
                            (1,),
                            mode=lax.GatherScatterMode.PROMISE_IN_BOUNDS)
                        rows[e, pl.ds(0, LANE)] = rows[e, pl.ds(0, LANE)] * we
                        rows[e, pl.ds(LANE, LANE)] = (
                            rows[e, pl.ds(LANE, LANE)] * we)
                    return c2

                lax.fori_loop(0, 8, gbody, 0)
                pltpu.sync_copy(rows, acc.at[dstbuf.at[j]], add=True)
                pltpu.sync_copy(cpay, cacc.at[dstbuf.at[j]], add=True)
            return carry

        lax.fori_loop(0, NROUT, chunk, 0)
        plsc.subcore_barrier()
        pltpu.sync_copy(acc.at[pl.ds(r0, RPT)], zbuf)
        pltpu.sync_copy(zbuf, acc_out.at[cid, pl.ds(r0, RPT)])
        pltpu.sync_copy(cacc.at[pl.ds(r0, RPT)], zbuf16)
        pltpu.sync_copy(zbuf16, cnt_out.at[cid, pl.ds(r0, RPT)])

    acc_out, cnt_out = sc_aggregate1(p1, src2d, dst2d, wgt, cpat)

    pa = acc_out[0, :N]
    pb = acc_out[1, :N]
    ca = cnt_out[0, :N]
    cb = cnt_out[1, :N]
    W2relp = jnp.zeros((H, HP), jnp.float32).at[:, :C].set(W2_rel)
    W2rootp = jnp.zeros((H, HP), jnp.float32).at[:, :C].set(W2_root)
    b2p = jnp.zeros((1, HP), jnp.float32).at[0, :C].set(b2)

    def b_body(pa_ref, pb_ref, ca_ref, cb_ref, r1_ref, wrel_ref, wroot_ref,
               b2_ref, p2_ref, r2_ref):
        s = pa_ref[...] + pb_ref[...]
        cnt = ca_ref[...][:, 0:1] + cb_ref[...][:, 0:1]
        mean = s / jnp.maximum(cnt, 1.0)
        h = jnp.maximum(mean + r1_ref[...], 0.0)
        p2_ref[...] = jnp.dot(h, wrel_ref[...],
                              preferred_element_type=jnp.float32)
        r2_ref[...] = jnp.dot(h, wroot_ref[...],
                              preferred_element_type=jnp.float32) + b2_ref[...]

    p2, r2 = pl.pallas_call(
        b_body,
        grid=(N // bn,),
        in_specs=[
            pl.BlockSpec((bn, H), lambda i: (i, 0)),
            pl.BlockSpec((bn, H), lambda i: (i, 0)),
            pl.BlockSpec((bn, HP), lambda i: (i, 0)),
            pl.BlockSpec((bn, HP), lambda i: (i, 0)),
            pl.BlockSpec((bn, H), lambda i: (i, 0)),
            pl.BlockSpec((H, HP), lambda i: (0, 0)),
            pl.BlockSpec((H, HP), lambda i: (0, 0)),
            pl.BlockSpec((1, HP), lambda i: (0, 0)),
        ],
        out_specs=[
            pl.BlockSpec((bn, HP), lambda i: (i, 0)),
            pl.BlockSpec((bn, HP), lambda i: (i, 0)),
        ],
        out_shape=[
            jax.ShapeDtypeStruct((N, HP), jnp.float32),
            jax.ShapeDtypeStruct((N, HP), jnp.float32),
        ],
    )(pa, pb, ca, cb, r1, W2relp, W2rootp, b2p)

    @functools.partial(
        pl.kernel,
        mesh=mesh,
        compiler_params=pltpu.CompilerParams(use_tc_tiling_on_sc=False),
        out_type=jax.ShapeDtypeStruct((2, NACC, HP), jnp.float32),
        scratch_types=[
            pltpu.VMEM((8, 128), jnp.int32),
            pltpu.VMEM((8, 128), jnp.int32),
            pltpu.VMEM((128, HP), jnp.float32),
            pltpu.VMEM((RPT, HP), jnp.float32),
            pltpu.VMEM_SHARED((NACC, HP), jnp.float32),
            pltpu.SemaphoreType.DMA,
        ],
    )
    def sc_aggregate2(p2_hbm, src_hbm, dst_hbm, acc_out2,
                      srcbuf, dstbuf, rows, zbuf, acc, sem):
        cid = lax.axis_index("c")
        sid = lax.axis_index("s")
        wid = cid * 16 + sid
        z16 = jnp.zeros((LANE,), jnp.float32)

        def zb(i, carry):
            zbuf[i, pl.ds(0, LANE)] = z16
            return carry

        lax.fori_loop(0, RPT, zb, 0)
        r0 = sid * RPT
        pltpu.sync_copy(zbuf, acc.at[pl.ds(r0, RPT)])
        plsc.subcore_barrier()

        rb = wid * NR

        def chunk(k, carry):
            pltpu.sync_copy(src_hbm.at[pl.ds(rb + k * 8, 8)], srcbuf)
            pltpu.sync_copy(dst_hbm.at[pl.ds(rb + k * 8, 8)], dstbuf)
            for j in range(8):
                pltpu.async_copy(p2_hbm.at[srcbuf.at[j]], rows, sem).wait()
                pltpu.sync_copy(rows, acc.at[dstbuf.at[j]], add=True)
            return carry

        lax.fori_loop(0, NROUT, chunk, 0)
        plsc.subcore_barrier()
        pltpu.sync_copy(acc.at[pl.ds(r0, RPT)], zbuf)
        pltpu.sync_copy(zbuf, acc_out2.at[cid, pl.ds(r0, RPT)])

    acc_out2 = sc_aggregate2(p2, src2d, dst2d)

    qa = acc_out2[0, :N]
    qb = acc_out2[1, :N]

    def c_body(qa_ref, qb_ref, r2_ref, ca_ref, cb_ref, o_ref):
        s2 = qa_ref[...] + qb_ref[...]
        cnt = ca_ref[...][:, 0:1] + cb_ref[...][:, 0:1]
        logits = s2 / jnp.maximum(cnt, 1.0) + r2_ref[...]
        col = lax.broadcasted_iota(jnp.int32, logits.shape, 1)
        valid = col < C
        neg = jnp.where(valid, logits, -jnp.inf)
        m = jnp.max(neg, axis=1, keepdims=True)
        ex = jnp.where(valid, jnp.exp(logits - m), 0.0)
        lse = jnp.log(jnp.sum(ex, axis=1, keepdims=True)) + m
        o_ref[...] = logits - lse

    out16 = pl.pallas_call(
        c_body,
        grid=(N // bn,),
        in_specs=[
            pl.BlockSpec((bn, HP), lambda i: (i, 0)),
            pl.BlockSpec((bn, HP), lambda i: (i, 0)),
            pl.BlockSpec((bn, HP), lambda i: (i, 0)),
            pl.BlockSpec((bn, HP), lambda i: (i, 0)),
            pl.BlockSpec((bn, HP), lambda i: (i, 0)),
        ],
        out_specs=pl.BlockSpec((bn, HP), lambda i: (i, 0)),
        out_shape=jax.ShapeDtypeStruct((N, HP), jnp.float32),
    )(qa, qb, r2, ca, cb)

    return out16[:, :C]
